# baseline (speedup 1.0000x reference)
import functools

import jax
import jax.numpy as jnp
from jax.experimental import pallas as pl
from jax.experimental.pallas import tpu as pltpu

B = 1024
T = 4
D_IN = 768
D = T * D_IN
S = 8192
K = 64
P0 = 4096

BT = 256
ST = 512
NB = B // BT
NS = S // ST
SEL_BT = 256


def _enc_body(x_ref, w_ref, b_ref, out_ref):
    out_ref[...] = jax.lax.dot_general(
        x_ref[...], w_ref[...], (((1,), (1,)), ((), ())),
        preferred_element_type=jnp.float32) + b_ref[...]


def _sel_body(pre_ref, thr_ref):
    q = jnp.maximum(pre_ref[...], 0.0)
    key = jax.lax.bitcast_convert_type(q, jnp.int32)

    def step(i, prefix):
        bit = 30 - i
        cand = prefix | (jnp.int32(1) << bit)
        cnt = jnp.sum((key >= cand).astype(jnp.int32), axis=1, keepdims=True)
        return jnp.where(cnt >= K, cand, prefix)

    prefix = jax.lax.fori_loop(
        0, 31, step, jnp.zeros((SEL_BT, 1), jnp.int32))
    thr = jax.lax.bitcast_convert_type(prefix, jnp.float32)
    thr_ref[...] = jnp.broadcast_to(thr, (SEL_BT, 128))


def _dec_body(pre_ref, thr_ref, w1_ref, w0_ref, x_ref, b0_ref, b1_ref,
              out_ref, acc0_ref, acc1_ref):
    b = pl.program_id(0)
    s = pl.program_id(1)

    @pl.when(s == 0)
    def _():
        acc0_ref[...] = jnp.zeros_like(acc0_ref)
        acc1_ref[...] = jnp.zeros_like(acc1_ref)

    q = jnp.maximum(pre_ref[...], 0.0)
    thr = thr_ref[:, :1]
    z = jnp.where(q >= thr, q, 0.0)

    acc1_ref[...] += jax.lax.dot_general(
        z, w1_ref[...], (((1,), (0,)), ((), ())),
        preferred_element_type=jnp.float32)

    @pl.when(s < P0 // ST)
    def _():
        acc0_ref[...] += jax.lax.dot_general(
            z, w0_ref[...], (((1,), (0,)), ((), ())),
            preferred_element_type=jnp.float32)

    @pl.when(s == NS - 1)
    def _():
        xb = x_ref[...]
        e1 = acc1_ref[...] + b1_ref[...] - xb
        e0 = acc0_ref[...] + b0_ref[...] - xb
        blk = (jnp.sum(e1 * e1) + jnp.sum(e0 * e0)).reshape(1, 1)

        @pl.when(b == 0)
        def _():
            out_ref[...] = jnp.zeros((1, 1), jnp.float32)
        out_ref[...] += blk


@jax.jit
def kernel(x, W_enc, b_enc, W_dec0, W_dec1, b_dec0, b_dec1):
    del W_enc
    x_f = x.reshape(B, D)
    w1_f = W_dec1.reshape(S, D)
    w0_f = W_dec0.reshape(P0, D)
    benc_f = b_enc.reshape(1, S)
    b0_f = b_dec0.reshape(1, D)
    b1_f = b_dec1.reshape(1, D)

    pre = pl.pallas_call(
        _enc_body,
        grid=(NB, NS),
        in_specs=[
            pl.BlockSpec((BT, D), lambda b, s: (b, 0)),
            pl.BlockSpec((ST, D), lambda b, s: (s, 0)),
            pl.BlockSpec((1, ST), lambda b, s: (0, s)),
        ],
        out_specs=pl.BlockSpec((BT, ST), lambda b, s: (b, s)),
        out_shape=jax.ShapeDtypeStruct((B, S), jnp.float32),
    )(x_f, w1_f, benc_f)

    thr = pl.pallas_call(
        _sel_body,
        grid=(B // SEL_BT,),
        in_specs=[pl.BlockSpec((SEL_BT, S), lambda i: (i, 0))],
        out_specs=pl.BlockSpec((SEL_BT, 128), lambda i: (i, 0)),
        out_shape=jax.ShapeDtypeStruct((B, 128), jnp.float32),
    )(pre)

    loss = pl.pallas_call(
        _dec_body,
        grid=(NB, NS),
        in_specs=[
            pl.BlockSpec((BT, ST), lambda b, s: (b, s)),
            pl.BlockSpec((BT, 128), lambda b, s: (b, 0)),
            pl.BlockSpec((ST, D), lambda b, s: (s, 0)),
            pl.BlockSpec((ST, D), lambda b, s: (jnp.minimum(s, P0 // ST - 1), 0)),
            pl.BlockSpec((BT, D), lambda b, s: (b, 0)),
            pl.BlockSpec((1, D), lambda b, s: (0, 0)),
            pl.BlockSpec((1, D), lambda b, s: (0, 0)),
        ],
        out_specs=pl.BlockSpec((1, 1), lambda b, s: (0, 0)),
        out_shape=jax.ShapeDtypeStruct((1, 1), jnp.float32),
        scratch_shapes=[
            pltpu.VMEM((BT, D), jnp.float32),
            pltpu.VMEM((BT, D), jnp.float32),
        ],
    )(pre, thr, w1_f, w0_f, x_f, b0_f, b1_f)

    return (loss[0, 0] / (B * T * 2)).astype(jnp.float32)


# speedup vs baseline: 1.0176x; 1.0176x over previous
import functools

import jax
import jax.numpy as jnp
from jax.experimental import pallas as pl
from jax.experimental.pallas import tpu as pltpu

B = 1024
T = 4
D_IN = 768
D = T * D_IN
S = 8192
K = 64
P0 = 4096

BT = 256
ST = 512
NB = B // BT
NS = S // ST
SEL_BT = 256


def _enc_body(x_ref, w_ref, b_ref, out_ref):
    out_ref[...] = jax.lax.dot_general(
        x_ref[...], w_ref[...], (((1,), (0,)), ((), ())),
        preferred_element_type=jnp.float32) + b_ref[...]


def _sel_body(pre_ref, thr_ref):
    q = jnp.maximum(pre_ref[...], 0.0)
    key = jax.lax.bitcast_convert_type(q, jnp.int32)

    def step(i, prefix):
        bit = 30 - i
        cand = prefix | (jnp.int32(1) << bit)
        cnt = jnp.sum((key >= cand).astype(jnp.int32), axis=1, keepdims=True)
        return jnp.where(cnt >= K, cand, prefix)

    prefix = jax.lax.fori_loop(
        0, 31, step, jnp.zeros((SEL_BT, 1), jnp.int32))
    thr = jax.lax.bitcast_convert_type(prefix, jnp.float32)
    thr_ref[...] = jnp.broadcast_to(thr, (SEL_BT, 128))


def _dec_body(pre_ref, thr_ref, w1_ref, w0_ref, x_ref, b0_ref, b1_ref,
              out_ref, acc0_ref, acc1_ref):
    b = pl.program_id(0)
    s = pl.program_id(1)

    @pl.when(s == 0)
    def _():
        acc0_ref[...] = jnp.zeros_like(acc0_ref)
        acc1_ref[...] = jnp.zeros_like(acc1_ref)

    q = jnp.maximum(pre_ref[...], 0.0)
    thr = thr_ref[:, :1]
    z = jnp.where(q >= thr, q, 0.0)

    acc1_ref[...] += jax.lax.dot_general(
        z, w1_ref[...], (((1,), (0,)), ((), ())),
        preferred_element_type=jnp.float32)

    @pl.when(s < P0 // ST)
    def _():
        acc0_ref[...] += jax.lax.dot_general(
            z, w0_ref[...], (((1,), (0,)), ((), ())),
            preferred_element_type=jnp.float32)

    @pl.when(s == NS - 1)
    def _():
        xb = x_ref[...]
        e1 = acc1_ref[...] + b1_ref[...] - xb
        e0 = acc0_ref[...] + b0_ref[...] - xb
        blk = (jnp.sum(e1 * e1) + jnp.sum(e0 * e0)).reshape(1, 1)

        @pl.when(b == 0)
        def _():
            out_ref[...] = jnp.zeros((1, 1), jnp.float32)
        out_ref[...] += blk


@jax.jit
def kernel(x, W_enc, b_enc, W_dec0, W_dec1, b_dec0, b_dec1):
    x_f = x.reshape(B, D)
    wenc_f = W_enc.reshape(D, S)
    w1_f = W_dec1.reshape(S, D)
    w0_f = W_dec0.reshape(P0, D)
    benc_f = b_enc.reshape(1, S)
    b0_f = b_dec0.reshape(1, D)
    b1_f = b_dec1.reshape(1, D)

    pre = pl.pallas_call(
        _enc_body,
        grid=(NB, NS),
        in_specs=[
            pl.BlockSpec((BT, D), lambda b, s: (b, 0)),
            pl.BlockSpec((D, ST), lambda b, s: (0, s)),
            pl.BlockSpec((1, ST), lambda b, s: (0, s)),
        ],
        out_specs=pl.BlockSpec((BT, ST), lambda b, s: (b, s)),
        out_shape=jax.ShapeDtypeStruct((B, S), jnp.float32),
    )(x_f, wenc_f, benc_f)

    thr = pl.pallas_call(
        _sel_body,
        grid=(B // SEL_BT,),
        in_specs=[pl.BlockSpec((SEL_BT, S), lambda i: (i, 0))],
        out_specs=pl.BlockSpec((SEL_BT, 128), lambda i: (i, 0)),
        out_shape=jax.ShapeDtypeStruct((B, 128), jnp.float32),
    )(pre)

    loss = pl.pallas_call(
        _dec_body,
        grid=(NB, NS),
        in_specs=[
            pl.BlockSpec((BT, ST), lambda b, s: (b, s)),
            pl.BlockSpec((BT, 128), lambda b, s: (b, 0)),
            pl.BlockSpec((ST, D), lambda b, s: (s, 0)),
            pl.BlockSpec((ST, D), lambda b, s: (jnp.minimum(s, P0 // ST - 1), 0)),
            pl.BlockSpec((BT, D), lambda b, s: (b, 0)),
            pl.BlockSpec((1, D), lambda b, s: (0, 0)),
            pl.BlockSpec((1, D), lambda b, s: (0, 0)),
        ],
        out_specs=pl.BlockSpec((1, 1), lambda b, s: (0, 0)),
        out_shape=jax.ShapeDtypeStruct((1, 1), jnp.float32),
        scratch_shapes=[
            pltpu.VMEM((BT, D), jnp.float32),
            pltpu.VMEM((BT, D), jnp.float32),
        ],
    )(pre, thr, w1_f, w0_f, x_f, b0_f, b1_f)

    return (loss[0, 0] / (B * T * 2)).astype(jnp.float32)


# speedup vs baseline: 1.6713x; 1.6424x over previous
import jax
import jax.numpy as jnp
from jax.experimental import pallas as pl
from jax.experimental.pallas import tpu as pltpu

B = 1024
T = 4
D_IN = 768
D = T * D_IN
S = 8192
K = 64
P0 = 4096

ST = 512
NS = S // ST
DST = 256
DNS = S // DST
DNS0 = P0 // DST
SEL_BT = 256


def _enc_body(x_ref, w_ref, b_ref, out_ref):
    out_ref[...] = jax.lax.dot_general(
        x_ref[...], w_ref[...], (((1,), (0,)), ((), ())),
        preferred_element_type=jnp.float32) + b_ref[...]


def _sel_body(pre_ref, thr_ref):
    q = jnp.maximum(pre_ref[...], 0.0)
    key = jax.lax.bitcast_convert_type(q, jnp.int32)

    def step(i, prefix):
        bit = 30 - i
        cand = prefix | (jnp.int32(1) << bit)
        cnt = jnp.sum((key >= cand).astype(jnp.int32), axis=1, keepdims=True)
        return jnp.where(cnt >= K, cand, prefix)

    prefix = jax.lax.fori_loop(
        0, 31, step, jnp.zeros((SEL_BT, 1), jnp.int32))
    thr = jax.lax.bitcast_convert_type(prefix, jnp.float32)
    thr_ref[...] = jnp.broadcast_to(thr, (SEL_BT, 128))


def _dec_body(pre_ref, thr_ref, w1_ref, w0_ref, x_ref, b0_ref, b1_ref,
              out_ref, acc0_ref, acc1_ref):
    s = pl.program_id(0)

    @pl.when(s == 0)
    def _():
        acc0_ref[...] = jnp.zeros_like(acc0_ref)
        acc1_ref[...] = jnp.zeros_like(acc1_ref)

    q = jnp.maximum(pre_ref[...], 0.0)
    thr = thr_ref[:, :1]
    z = jnp.where(q >= thr, q, 0.0)

    acc1_ref[...] += jax.lax.dot_general(
        z, w1_ref[...], (((1,), (1,)), ((), ())),
        preferred_element_type=jnp.float32)

    @pl.when(s < DNS0)
    def _():
        acc0_ref[...] += jax.lax.dot_general(
            z, w0_ref[...], (((1,), (0,)), ((), ())),
            preferred_element_type=jnp.float32)

    @pl.when(s == DNS - 1)
    def _():
        xb = x_ref[...]
        e1 = acc1_ref[...] + b1_ref[...] - xb
        e0 = acc0_ref[...] + b0_ref[...] - xb
        out_ref[...] = (jnp.sum(e1 * e1) + jnp.sum(e0 * e0)).reshape(1, 1)


@jax.jit
def kernel(x, W_enc, b_enc, W_dec0, W_dec1, b_dec0, b_dec1):
    del W_dec1
    x_f = x.reshape(B, D)
    wenc_f = W_enc.reshape(D, S)
    w0_f = W_dec0.reshape(P0, D)
    benc_f = b_enc.reshape(1, S)
    b0_f = b_dec0.reshape(1, D)
    b1_f = b_dec1.reshape(1, D)

    pre = pl.pallas_call(
        _enc_body,
        grid=(NS,),
        in_specs=[
            pl.BlockSpec((B, D), lambda s: (0, 0)),
            pl.BlockSpec((D, ST), lambda s: (0, s)),
            pl.BlockSpec((1, ST), lambda s: (0, s)),
        ],
        out_specs=pl.BlockSpec((B, ST), lambda s: (0, s)),
        out_shape=jax.ShapeDtypeStruct((B, S), jnp.float32),
    )(x_f, wenc_f, benc_f)

    thr = pl.pallas_call(
        _sel_body,
        grid=(B // SEL_BT,),
        in_specs=[pl.BlockSpec((SEL_BT, S), lambda i: (i, 0))],
        out_specs=pl.BlockSpec((SEL_BT, 128), lambda i: (i, 0)),
        out_shape=jax.ShapeDtypeStruct((B, 128), jnp.float32),
    )(pre)

    loss = pl.pallas_call(
        _dec_body,
        grid=(DNS,),
        in_specs=[
            pl.BlockSpec((B, DST), lambda s: (0, s)),
            pl.BlockSpec((B, 128), lambda s: (0, 0)),
            pl.BlockSpec((D, DST), lambda s: (0, s)),
            pl.BlockSpec((DST, D), lambda s: (jnp.minimum(s, DNS0 - 1), 0)),
            pl.BlockSpec((B, D), lambda s: (0, 0)),
            pl.BlockSpec((1, D), lambda s: (0, 0)),
            pl.BlockSpec((1, D), lambda s: (0, 0)),
        ],
        out_specs=pl.BlockSpec((1, 1), lambda s: (0, 0)),
        out_shape=jax.ShapeDtypeStruct((1, 1), jnp.float32),
        scratch_shapes=[
            pltpu.VMEM((B, D), jnp.float32),
            pltpu.VMEM((B, D), jnp.float32),
        ],
    )(pre, thr, wenc_f, w0_f, x_f, b0_f, b1_f)

    return (loss[0, 0] / (B * T * 2)).astype(jnp.float32)
